# baseline (speedup 1.0000x reference)
import jax
import jax.numpy as jnp
from jax.experimental import pallas as pl
from jax.experimental.pallas import tpu as pltpu

_TOKENS = 8192
_DIM = 2048
_EXPERTS = 64
_BT = 1024
_BTS = 256
_NMAIN = 7
_NTAIL = 4
_TAIL_BLK0 = _NMAIN * _BT // _BTS


def _softmax(logits):
    m = jnp.max(logits, axis=1, keepdims=True)
    e = jnp.exp(logits - m)
    return e / jnp.sum(e, axis=1, keepdims=True)


def _gate_kernel(xa_ref, xb_ref, w_ref, out_ref):
    i = pl.program_id(0)
    w = w_ref[...]
    dn = (((1,), (1,)), ((), ()))

    @pl.when(i < _NMAIN)
    def _():
        out_ref[...] = _softmax(jax.lax.dot_general(
            xa_ref[...], w, dimension_numbers=dn,
            preferred_element_type=jnp.float32))

    @pl.when(i >= _NMAIN)
    def _():
        j = i - _NMAIN
        gates = _softmax(jax.lax.dot_general(
            xb_ref[...], w, dimension_numbers=dn,
            preferred_element_type=jnp.float32))
        out_ref[pl.ds(j * _BTS, _BTS), :] = gates


def kernel(inputs, wg):
    return pl.pallas_call(
        _gate_kernel,
        grid=(_NMAIN + _NTAIL,),
        in_specs=[
            pl.BlockSpec((_BT, _DIM),
                         lambda i: (jnp.minimum(i, _NMAIN - 1), 0)),
            pl.BlockSpec((_BTS, _DIM),
                         lambda i: (_TAIL_BLK0
                                    + jnp.clip(i - _NMAIN, 0, _NTAIL - 1), 0)),
            pl.BlockSpec((_EXPERTS, _DIM), lambda i: (0, 0)),
        ],
        out_specs=pl.BlockSpec((_BT, _EXPERTS),
                               lambda i: (jnp.minimum(i, _NMAIN), 0)),
        out_shape=jax.ShapeDtypeStruct((_TOKENS, _EXPERTS), jnp.float32),
        compiler_params=pltpu.CompilerParams(
            dimension_semantics=("arbitrary",)),
    )(inputs, inputs, wg)


# speedup vs baseline: 1.1276x; 1.1276x over previous
import jax
import jax.numpy as jnp
from jax.experimental import pallas as pl
from jax.experimental.pallas import tpu as pltpu

_TOKENS = 8192
_DIM = 2048
_EXPERTS = 64
_BT = 1024


def _gate_kernel(x_ref, w_ref, out_ref):
    logits = jax.lax.dot_general(
        x_ref[...], w_ref[...],
        dimension_numbers=(((1,), (1,)), ((), ())),
        preferred_element_type=jnp.float32)
    m = jnp.max(logits, axis=1, keepdims=True)
    e = jnp.exp(logits - m)
    out_ref[...] = e / jnp.sum(e, axis=1, keepdims=True)


def kernel(inputs, wg):
    return pl.pallas_call(
        _gate_kernel,
        grid=(_TOKENS // _BT,),
        in_specs=[
            pl.BlockSpec((_BT, _DIM), lambda i: (i, 0)),
            pl.BlockSpec((_EXPERTS, _DIM), lambda i: (0, 0)),
        ],
        out_specs=pl.BlockSpec((_BT, _EXPERTS), lambda i: (i, 0)),
        out_shape=jax.ShapeDtypeStruct((_TOKENS, _EXPERTS), jnp.float32),
        compiler_params=pltpu.CompilerParams(
            dimension_semantics=("parallel",),
            disable_bounds_checks=True,
            disable_semaphore_checks=True),
    )(inputs, wg)
